# baseline (speedup 1.0000x reference)
import functools

import jax
import jax.numpy as jnp
from jax import lax
from jax.experimental import pallas as pl
from jax.experimental.pallas import tpu as pltpu
from jax.experimental.pallas import tpu_sc as plsc

B = 4096
F = 26
D = 32
NFLAT = B * F
NC, NS = 2, 16
NW = NC * NS
PER_W = NFLAT // NW

BN_EPS = 1e-5

_mesh = plsc.VectorSubcoreMesh(core_axis_name="c", subcore_axis_name="s")


@functools.partial(
    pl.kernel,
    mesh=_mesh,
    out_type=jax.ShapeDtypeStruct((NFLAT,), jnp.float32),
    scratch_types=[
        pltpu.VMEM((PER_W,), jnp.int32),
        pltpu.VMEM((PER_W,), jnp.float32),
        pltpu.SemaphoreType.DMA,
    ],
    compiler_params=pltpu.CompilerParams(use_tc_tiling_on_sc=False,
                                         skip_device_barrier=True),
)
def _sc_o1_gather(idx_hbm, o1_tab, o1_out, idx_v, o1_v, sem):
    wid = lax.axis_index("s") * NC + lax.axis_index("c")
    base = wid * PER_W
    pltpu.sync_copy(idx_hbm.at[pl.ds(base, PER_W)], idx_v)
    pltpu.async_copy(o1_tab.at[idx_v], o1_v, sem).wait()
    pltpu.sync_copy(o1_v, o1_out.at[pl.ds(base, PER_W)])


BB = 512
BLK1 = 256
BLK2 = 256


def _bn_relu(h, g, bt, ones_b):
    inv_b = 1.0 / B
    mu = jnp.dot(ones_b, h, preferred_element_type=jnp.float32) * inv_b
    m2 = jnp.dot(ones_b, h * h, preferred_element_type=jnp.float32) * inv_b
    var = m2 - mu * mu
    h = (h - mu) / jnp.sqrt(var + BN_EPS) * g + bt
    return jnp.maximum(h, 0.0)


def _tc_body(emb_ref, o1v_ref, W1_ref, b1_ref, g1_ref, bt1_ref,
             W2_ref, b2_ref, g2_ref, bt2_ref, W3_ref, b3_ref,
             W4_ref, b4_ref, out_ref, embbf_s, h1_s, h2_s):
    bf = jnp.bfloat16
    ones_b = jnp.ones((1, B), jnp.float32)
    rows = jax.lax.broadcasted_iota(jnp.int32, (F * D, D), 0)
    cols = jax.lax.broadcasted_iota(jnp.int32, (F * D, D), 1)
    S = jnp.where(rows % D == cols, 1.0, 0.0).astype(jnp.float32)
    ones_fd = jnp.ones((F * D, 1), jnp.float32)
    ones_d = jnp.ones((D, 1), jnp.float32)
    ones_f = jnp.ones((F, 1), jnp.float32)

    for i in range(B // BB):
        r = pl.ds(i * BB, BB)
        e = emb_ref[r, :]
        s = jnp.dot(e, S, preferred_element_type=jnp.float32)
        ssq = jnp.dot(e * e, ones_fd, preferred_element_type=jnp.float32)
        sqs = jnp.dot(s * s, ones_d, preferred_element_type=jnp.float32)
        o1 = jnp.dot(o1v_ref[r, :], ones_f, preferred_element_type=jnp.float32)
        out_ref[r, :] = o1 + 0.5 * (sqs - ssq)
        embbf_s[r, :] = e.astype(bf)

    for j in range(1024 // BLK1):
        cbl = pl.ds(j * BLK1, BLK1)
        h = jnp.dot(embbf_s[...], W1_ref[:, cbl],
                    preferred_element_type=jnp.float32) + b1_ref[:, cbl]
        h1_s[:, cbl] = _bn_relu(h, g1_ref[:, cbl], bt1_ref[:, cbl],
                                ones_b).astype(bf)

    for j in range(512 // BLK2):
        cbl = pl.ds(j * BLK2, BLK2)
        h = jnp.dot(h1_s[...], W2_ref[:, cbl],
                    preferred_element_type=jnp.float32) + b2_ref[:, cbl]
        h2_s[:, cbl] = _bn_relu(h, g2_ref[:, cbl], bt2_ref[:, cbl],
                                ones_b).astype(bf)

    h3 = jnp.dot(h2_s[...], W3_ref[...],
                 preferred_element_type=jnp.float32) + b3_ref[...]
    dnn = jnp.dot(h3.astype(bf), W4_ref[...],
                  preferred_element_type=jnp.float32) + b4_ref[...]
    out_ref[...] = out_ref[...] + dnn


def kernel(x, cat_embed, o1_table, W1, b1, g1, bt1, W2, b2, g2, bt2,
           W3, b3, W4, b4):
    bf = jnp.bfloat16
    emb2d = cat_embed.at[x].get(mode="promise_in_bounds").reshape(B, F * D)
    idx = x.reshape(-1).astype(jnp.int32)
    o1v = _sc_o1_gather(idx, o1_table[:, 0]).reshape(B, F)
    out = pl.pallas_call(
        _tc_body,
        out_shape=jax.ShapeDtypeStruct((B, 1), jnp.float32),
        scratch_shapes=[
            pltpu.VMEM((B, F * D), bf),
            pltpu.VMEM((B, 1024), bf),
            pltpu.VMEM((B, 512), bf),
        ],
    )(emb2d, o1v, W1.astype(bf), b1.reshape(1, -1), g1.reshape(1, -1),
      bt1.reshape(1, -1), W2.astype(bf), b2.reshape(1, -1),
      g2.reshape(1, -1), bt2.reshape(1, -1), W3.astype(bf),
      b3.reshape(1, -1), W4.astype(bf), b4.reshape(1, -1))
    return out


# speedup vs baseline: 1.3757x; 1.3757x over previous
import jax
import jax.numpy as jnp
from jax.experimental import pallas as pl
from jax.experimental.pallas import tpu as pltpu

B = 4096
F = 26
D = 32

BN_EPS = 1e-5

BB = 512
BLK1 = 256
BLK2 = 256


def _bn_relu(h, g, bt, ones_b):
    inv_b = 1.0 / B
    mu = jnp.dot(ones_b, h, preferred_element_type=jnp.float32) * inv_b
    m2 = jnp.dot(ones_b, h * h, preferred_element_type=jnp.float32) * inv_b
    var = m2 - mu * mu
    h = (h - mu) / jnp.sqrt(var + BN_EPS) * g + bt
    return jnp.maximum(h, 0.0)


def _tc_body(emb_ref, o1v_ref, W1_ref, b1_ref, g1_ref, bt1_ref,
             W2_ref, b2_ref, g2_ref, bt2_ref, W3_ref, b3_ref,
             W4_ref, b4_ref, out_ref, embbf_s, h1_s, h2_s):
    bf = jnp.bfloat16
    ones_b = jnp.ones((1, B), jnp.float32)
    rows = jax.lax.broadcasted_iota(jnp.int32, (F * D, D), 0)
    cols = jax.lax.broadcasted_iota(jnp.int32, (F * D, D), 1)
    S = jnp.where(rows % D == cols, 1.0, 0.0).astype(jnp.float32)
    ones_fd = jnp.ones((F * D, 1), jnp.float32)
    ones_d = jnp.ones((D, 1), jnp.float32)
    ones_f = jnp.ones((F, 1), jnp.float32)

    for i in range(B // BB):
        r = pl.ds(i * BB, BB)
        e = emb_ref[r, :]
        s = jnp.dot(e, S, preferred_element_type=jnp.float32)
        ssq = jnp.dot(e * e, ones_fd, preferred_element_type=jnp.float32)
        sqs = jnp.dot(s * s, ones_d, preferred_element_type=jnp.float32)
        o1 = jnp.dot(o1v_ref[r, :], ones_f, preferred_element_type=jnp.float32)
        out_ref[r, :] = o1 + 0.5 * (sqs - ssq)
        embbf_s[r, :] = e.astype(bf)

    for j in range(1024 // BLK1):
        cbl = pl.ds(j * BLK1, BLK1)
        h = jnp.dot(embbf_s[...], W1_ref[:, cbl],
                    preferred_element_type=jnp.float32) + b1_ref[:, cbl]
        h1_s[:, cbl] = _bn_relu(h, g1_ref[:, cbl], bt1_ref[:, cbl],
                                ones_b).astype(bf)

    for j in range(512 // BLK2):
        cbl = pl.ds(j * BLK2, BLK2)
        h = jnp.dot(h1_s[...], W2_ref[:, cbl],
                    preferred_element_type=jnp.float32) + b2_ref[:, cbl]
        h2_s[:, cbl] = _bn_relu(h, g2_ref[:, cbl], bt2_ref[:, cbl],
                                ones_b).astype(bf)

    h3 = jnp.dot(h2_s[...], W3_ref[...],
                 preferred_element_type=jnp.float32) + b3_ref[...]
    dnn = jnp.dot(h3.astype(bf), W4_ref[...],
                  preferred_element_type=jnp.float32) + b4_ref[...]
    out_ref[...] = out_ref[...] + dnn


def kernel(x, cat_embed, o1_table, W1, b1, g1, bt1, W2, b2, g2, bt2,
           W3, b3, W4, b4):
    bf = jnp.bfloat16
    emb2d = cat_embed.at[x].get(mode="promise_in_bounds").reshape(B, F * D)
    o1v = o1_table.at[x].get(mode="promise_in_bounds").reshape(B, F)
    out = pl.pallas_call(
        _tc_body,
        out_shape=jax.ShapeDtypeStruct((B, 1), jnp.float32),
        scratch_shapes=[
            pltpu.VMEM((B, F * D), bf),
            pltpu.VMEM((B, 1024), bf),
            pltpu.VMEM((B, 512), bf),
        ],
    )(emb2d, o1v, W1.astype(bf), b1.reshape(1, -1), g1.reshape(1, -1),
      bt1.reshape(1, -1), W2.astype(bf), b2.reshape(1, -1),
      g2.reshape(1, -1), bt2.reshape(1, -1), W3.astype(bf),
      b3.reshape(1, -1), W4.astype(bf), b4.reshape(1, -1))
    return out
